# baseline (speedup 1.0000x reference)
import functools

import jax
import jax.numpy as jnp
from jax import lax
from jax.experimental import pallas as pl
from jax.experimental.pallas import tpu as pltpu
from jax.experimental.pallas import tpu_sc as plsc

A = 2048
D = 768
E = 8
F = 1024
B = 256
NBP = A // B + (E - 1)
NPAD = NBP * B
NBV = -(-NBP // 16)
NC, NS = 2, 16
NT = NC * NS
TPT = A // NT
NVR = TPT // 16
CPR = D // 16

_SC_MESH = dict(core_axis_name="c", subcore_axis_name="s",
                num_cores=NC, num_subcores=NS)


def _router_body(x_ref, r_ref, eid_ref, hist_ref, xg_ref):
    xb = x_ref[...]
    scores = jnp.dot(xb, r_ref[...], preferred_element_type=jnp.float32)
    eid = jnp.argmax(scores, axis=1).astype(jnp.int32)
    gate = jax.nn.sigmoid(jnp.max(scores, axis=1))
    eid_ref[...] = eid.reshape(1, A)
    sub = eid.reshape(NT, TPT, 1)
    oh = (sub == lax.broadcasted_iota(jnp.int32, (1, 1, 16), 2))
    hist_ref[...] = jnp.sum(oh.astype(jnp.int32), axis=1)
    xg_ref[...] = xb * gate[:, None]


def _router(x, router_DE):
    return pl.pallas_call(
        _router_body,
        out_specs=[
            pl.BlockSpec((1, A), lambda: (0, 0)),
            pl.BlockSpec((NT, 16), lambda: (0, 0)),
            pl.BlockSpec((A, D), lambda: (0, 0)),
        ],
        out_shape=[
            jax.ShapeDtypeStruct((1, A), jnp.int32),
            jax.ShapeDtypeStruct((NT, 16), jnp.int32),
            jax.ShapeDtypeStruct((A, D), jnp.float32),
        ],
    )(x, router_DE)


def _shared_body(x_ref, sw1_ref, sw3_ref, sw2_ref, so_ref):
    xb16 = x_ref[...].astype(jnp.bfloat16)
    h1 = jnp.dot(xb16, sw1_ref[...].astype(jnp.bfloat16),
                 preferred_element_type=jnp.float32)
    h3 = jnp.dot(xb16, sw3_ref[...].astype(jnp.bfloat16),
                 preferred_element_type=jnp.float32)
    mid = (h1 * jax.nn.sigmoid(h1) * h3).astype(jnp.bfloat16)
    so_ref[...] = jnp.dot(mid, sw2_ref[...].astype(jnp.bfloat16),
                          preferred_element_type=jnp.float32)


def _shared(x, sw1, sw3, sw2):
    nb = A // B
    return pl.pallas_call(
        _shared_body,
        grid=(nb,),
        in_specs=[
            pl.BlockSpec((B, D), lambda i: (i, 0)),
            pl.BlockSpec((D, F), lambda i: (0, 0)),
            pl.BlockSpec((D, F), lambda i: (0, 0)),
            pl.BlockSpec((F, D), lambda i: (0, 0)),
        ],
        out_specs=pl.BlockSpec((B, D), lambda i: (i, 0)),
        out_shape=jax.ShapeDtypeStruct((A, D), jnp.float32),
    )(x, sw1, sw3, sw2)


def _sort_scatter_body(eid_hbm, xg_hbm, hist_hbm, xs_hbm, pos_hbm, beid_hbm,
                       fo_hbm, slot_hbm,
                       eid_v, rows_v, pos_v, allhist_v, beid_v, fo_v, slot_v,
                       sem):
    wid = lax.axis_index("s") * NC + lax.axis_index("c")
    base = wid * TPT
    pltpu.sync_copy(eid_hbm.at[0, pl.ds(base, TPT)], eid_v)
    pltpu.sync_copy(hist_hbm, allhist_v)
    lane = lax.iota(jnp.int32, 16)
    tot = jnp.zeros((16,), jnp.int32)
    pre = jnp.zeros((16,), jnp.int32)
    for t in range(NT):
        row = allhist_v[t, :]
        tot = tot + row
        pre = pre + jnp.where(jnp.int32(t) < wid, row, jnp.int32(0))
    padded = ((tot + (B - 1)) // B) * B
    cum = plsc.cumsum(padded)
    off = cum - padded
    start = off + pre
    cums = [jnp.sum(jnp.where(lane == e, cum, jnp.int32(0))) for e in range(E)]
    for v in range(NBV):
        bstart = (lane + 16 * v) * B
        beid = jnp.zeros((16,), jnp.int32)
        for e in range(E):
            beid = beid + jnp.where(bstart >= cums[e], 1, 0).astype(jnp.int32)
        beid_v[pl.ds(16 * v, 16)] = jnp.minimum(beid, E - 1)

    present = tot > 0
    rank = plsc.cumsum(jnp.where(present, 1, 0).astype(jnp.int32)) - 1
    fo_v[...] = jnp.where(present, off // B, -1)
    slot_v[...] = jnp.where(present, rank % 4, 0)

    @pl.when(wid == 0)
    def _():
        pltpu.sync_copy(beid_v, beid_hbm)
        pltpu.sync_copy(fo_v, fo_hbm)
        pltpu.sync_copy(slot_v, slot_hbm)

    starts = [jnp.sum(jnp.where(lane == e, start, jnp.int32(0)))
              for e in range(E)]
    for k in range(NVR):
        ev = eid_v[pl.ds(k * 16, 16)]
        posk = jnp.zeros((16,), jnp.int32)
        for e in range(E):
            m = ev == e
            mi = jnp.where(m, 1, 0).astype(jnp.int32)
            c = plsc.cumsum(mi)
            posk = jnp.where(m, starts[e] + c - 1, posk)
            starts[e] = starts[e] + jnp.sum(mi)
        pos_v[pl.ds(k * 16, 16)] = posk
    pltpu.sync_copy(pos_v, pos_hbm.at[pl.ds(base, TPT)])
    pltpu.sync_copy(xg_hbm.at[pl.ds(base, TPT)], rows_v)
    pltpu.async_copy(rows_v, xs_hbm.at[pos_v], sem).wait()


def _sort_scatter(eid, xg, hist):
    return pl.kernel(
        _sort_scatter_body,
        out_type=(
            jax.ShapeDtypeStruct((NPAD, D), jnp.float32),
            jax.ShapeDtypeStruct((A,), jnp.int32),
            jax.ShapeDtypeStruct((NBV * 16,), jnp.int32),
            jax.ShapeDtypeStruct((16,), jnp.int32),
            jax.ShapeDtypeStruct((16,), jnp.int32),
        ),
        mesh=plsc.VectorSubcoreMesh(**_SC_MESH),
        compiler_params=pltpu.CompilerParams(needs_layout_passes=False),
        scratch_types=[
            pltpu.VMEM((TPT,), jnp.int32),
            pltpu.VMEM((TPT, D), jnp.float32),
            pltpu.VMEM((TPT,), jnp.int32),
            pltpu.VMEM((NT, 16), jnp.int32),
            pltpu.VMEM((NBV * 16,), jnp.int32),
            pltpu.VMEM((16,), jnp.int32),
            pltpu.VMEM((16,), jnp.int32),
            pltpu.SemaphoreType.DMA,
        ],
    )(eid, xg, hist)


_LA = 3
_NBUF = 4


def _grouped_body(beid_ref, fo_ref, slot_ref, xs_ref, w1_hbm, w3_hbm, w2_hbm,
                  ys_ref, w1b, w3b, w2b, sems):
    i = pl.program_id(0)

    def w_copies(e_idx, s_idx):
        return (
            pltpu.make_async_copy(w1_hbm.at[e_idx], w1b.at[s_idx],
                                  sems.at[s_idx, 0]),
            pltpu.make_async_copy(w3_hbm.at[e_idx], w3b.at[s_idx],
                                  sems.at[s_idx, 1]),
            pltpu.make_async_copy(w2_hbm.at[e_idx], w2b.at[s_idx],
                                  sems.at[s_idx, 2]),
        )

    for ee in range(E):
        fo_e = fo_ref[ee]
        sl_e = slot_ref[ee]

        @pl.when((fo_e >= 0) & (i == jnp.maximum(fo_e - _LA, 0)))
        def _():
            for c in w_copies(ee, sl_e):
                c.start()

        @pl.when(fo_e == i)
        def _():
            for c in w_copies(ee, sl_e):
                c.wait()

    e = beid_ref[i]
    sl = slot_ref[e]
    xb = xs_ref[...].astype(jnp.bfloat16)
    w1c = w1b[pl.ds(sl, 1)][0].astype(jnp.bfloat16)
    w3c = w3b[pl.ds(sl, 1)][0].astype(jnp.bfloat16)
    w2c = w2b[pl.ds(sl, 1)][0].astype(jnp.bfloat16)
    h1 = jnp.dot(xb, w1c, preferred_element_type=jnp.float32)
    h3 = jnp.dot(xb, w3c, preferred_element_type=jnp.float32)
    mid = (h1 * jax.nn.sigmoid(h1) * h3).astype(jnp.bfloat16)
    ys_ref[...] = jnp.dot(mid, w2c, preferred_element_type=jnp.float32)


def _grouped(beid, fo, slot, xs, w1, w3, w2):
    grid_spec = pltpu.PrefetchScalarGridSpec(
        num_scalar_prefetch=3,
        grid=(NBP,),
        in_specs=[
            pl.BlockSpec((B, D), lambda i, be, fo_, sl: (i, 0)),
            pl.BlockSpec(memory_space=pltpu.MemorySpace.HBM),
            pl.BlockSpec(memory_space=pltpu.MemorySpace.HBM),
            pl.BlockSpec(memory_space=pltpu.MemorySpace.HBM),
        ],
        out_specs=pl.BlockSpec((B, D), lambda i, be, fo_, sl: (i, 0)),
        scratch_shapes=[
            pltpu.VMEM((_NBUF, D, F), jnp.float32),
            pltpu.VMEM((_NBUF, D, F), jnp.float32),
            pltpu.VMEM((_NBUF, F, D), jnp.float32),
            pltpu.SemaphoreType.DMA((_NBUF, 3)),
        ],
    )
    return pl.pallas_call(
        _grouped_body,
        grid_spec=grid_spec,
        out_shape=jax.ShapeDtypeStruct((NPAD, D), jnp.float32),
    )(beid, fo, slot, xs, w1, w3, w2)


def _merge_body(pos_hbm, shared_hbm, ys_hbm, out_hbm, pos_v, rows_v, sh_v, sem):
    wid = lax.axis_index("s") * NC + lax.axis_index("c")
    base = wid * TPT
    pltpu.sync_copy(pos_hbm.at[pl.ds(base, TPT)], pos_v)
    pltpu.async_copy(ys_hbm.at[pos_v], rows_v, sem).wait()
    pltpu.sync_copy(shared_hbm.at[pl.ds(base, TPT)], sh_v)

    def _add_row(r, carry):
        for c in range(CPR):
            rows_v[r, pl.ds(c * 16, 16)] = (rows_v[r, pl.ds(c * 16, 16)]
                                            + sh_v[r, pl.ds(c * 16, 16)])
        return carry

    lax.fori_loop(0, TPT, _add_row, 0)
    pltpu.sync_copy(rows_v, out_hbm.at[pl.ds(base, TPT)])


def _merge(pos, shared, ys):
    return pl.kernel(
        _merge_body,
        out_type=jax.ShapeDtypeStruct((A, D), jnp.float32),
        mesh=plsc.VectorSubcoreMesh(**_SC_MESH),
        compiler_params=pltpu.CompilerParams(needs_layout_passes=False),
        scratch_types=[
            pltpu.VMEM((TPT,), jnp.int32),
            pltpu.VMEM((TPT, D), jnp.float32),
            pltpu.VMEM((TPT, D), jnp.float32),
            pltpu.SemaphoreType.DMA,
        ],
    )(pos, shared, ys)


def kernel(x_bsD, router_DE, w1, w3, w2, sw1, sw3, sw2):
    b, s, d = x_bsD.shape
    x = x_bsD.reshape(-1, d)
    eid2, hist, xg = _router(x, router_DE)
    xs, pos, beid, fo, slot = _sort_scatter(eid2, xg, hist)
    shared = _shared(x, sw1, sw3, sw2)
    ys = _grouped(beid, fo, slot, xs, w1, w3, w2)
    out = _merge(pos, shared, ys)
    return out.reshape(b, s, d)


# speedup vs baseline: 1.0133x; 1.0133x over previous
import functools

import jax
import jax.numpy as jnp
from jax import lax
from jax.experimental import pallas as pl
from jax.experimental.pallas import tpu as pltpu
from jax.experimental.pallas import tpu_sc as plsc

A = 2048
D = 768
E = 8
F = 1024
B = 256
NBP = A // B + (E - 1)
NPAD = NBP * B
NBV = -(-NBP // 16)
NC, NS = 2, 16
NT = NC * NS
TPT = A // NT
NVR = TPT // 16
CPR = D // 16

_SC_MESH = dict(core_axis_name="c", subcore_axis_name="s",
                num_cores=NC, num_subcores=NS)


def _router_body(x_ref, r_ref, eid_ref, hist_ref, xg_ref):
    xb = x_ref[...]
    scores = jnp.dot(xb, r_ref[...], preferred_element_type=jnp.float32)
    eid = jnp.argmax(scores, axis=1).astype(jnp.int32)
    gate = jax.nn.sigmoid(jnp.max(scores, axis=1))
    eid_ref[...] = eid.reshape(1, A)
    sub = eid.reshape(NT, TPT, 1)
    oh = (sub == lax.broadcasted_iota(jnp.int32, (1, 1, 16), 2))
    hist_ref[...] = jnp.sum(oh.astype(jnp.int32), axis=1)
    xg_ref[...] = xb * gate[:, None]


def _router(x, router_DE):
    return pl.pallas_call(
        _router_body,
        out_specs=[
            pl.BlockSpec((1, A), lambda: (0, 0)),
            pl.BlockSpec((NT, 16), lambda: (0, 0)),
            pl.BlockSpec((A, D), lambda: (0, 0)),
        ],
        out_shape=[
            jax.ShapeDtypeStruct((1, A), jnp.int32),
            jax.ShapeDtypeStruct((NT, 16), jnp.int32),
            jax.ShapeDtypeStruct((A, D), jnp.float32),
        ],
    )(x, router_DE)


def _shared_body(x_ref, sw1_ref, sw3_ref, sw2_ref, so_ref):
    xb16 = x_ref[...].astype(jnp.bfloat16)
    h1 = jnp.dot(xb16, sw1_ref[...].astype(jnp.bfloat16),
                 preferred_element_type=jnp.float32)
    h3 = jnp.dot(xb16, sw3_ref[...].astype(jnp.bfloat16),
                 preferred_element_type=jnp.float32)
    mid = (h1 * jax.nn.sigmoid(h1) * h3).astype(jnp.bfloat16)
    so_ref[...] = jnp.dot(mid, sw2_ref[...].astype(jnp.bfloat16),
                          preferred_element_type=jnp.float32)


def _shared(x, sw1, sw3, sw2):
    nb = A // B
    return pl.pallas_call(
        _shared_body,
        grid=(nb,),
        in_specs=[
            pl.BlockSpec((B, D), lambda i: (i, 0)),
            pl.BlockSpec((D, F), lambda i: (0, 0)),
            pl.BlockSpec((D, F), lambda i: (0, 0)),
            pl.BlockSpec((F, D), lambda i: (0, 0)),
        ],
        out_specs=pl.BlockSpec((B, D), lambda i: (i, 0)),
        out_shape=jax.ShapeDtypeStruct((A, D), jnp.float32),
    )(x, sw1, sw3, sw2)


def _sort_scatter_body(eid_hbm, xg_hbm, hist_hbm, xs_hbm, pos_hbm, beid_hbm,
                       fo_hbm, slot_hbm,
                       eid_v, rows_v, pos_v, allhist_v, beid_v, fo_v, slot_v,
                       sem):
    wid = lax.axis_index("s") * NC + lax.axis_index("c")
    base = wid * TPT
    pltpu.sync_copy(eid_hbm.at[0, pl.ds(base, TPT)], eid_v)
    pltpu.sync_copy(hist_hbm, allhist_v)
    lane = lax.iota(jnp.int32, 16)
    tot = jnp.zeros((16,), jnp.int32)
    pre = jnp.zeros((16,), jnp.int32)
    for t in range(NT):
        row = allhist_v[t, :]
        tot = tot + row
        pre = pre + jnp.where(jnp.int32(t) < wid, row, jnp.int32(0))
    padded = ((tot + (B - 1)) // B) * B
    cum = plsc.cumsum(padded)
    off = cum - padded
    start = off + pre
    cums = [jnp.sum(jnp.where(lane == e, cum, jnp.int32(0))) for e in range(E)]
    for v in range(NBV):
        bstart = (lane + 16 * v) * B
        beid = jnp.zeros((16,), jnp.int32)
        for e in range(E):
            beid = beid + jnp.where(bstart >= cums[e], 1, 0).astype(jnp.int32)
        beid_v[pl.ds(16 * v, 16)] = jnp.minimum(beid, E - 1)

    present = tot > 0
    rank = plsc.cumsum(jnp.where(present, 1, 0).astype(jnp.int32)) - 1
    fo_v[...] = jnp.where(present, off // B, -1)
    slot_v[...] = jnp.where(present, rank % 3, 0)

    @pl.when(wid == 0)
    def _():
        pltpu.sync_copy(beid_v, beid_hbm)
        pltpu.sync_copy(fo_v, fo_hbm)
        pltpu.sync_copy(slot_v, slot_hbm)

    starts = [jnp.sum(jnp.where(lane == e, start, jnp.int32(0)))
              for e in range(E)]
    for k in range(NVR):
        ev = eid_v[pl.ds(k * 16, 16)]
        posk = jnp.zeros((16,), jnp.int32)
        for e in range(E):
            m = ev == e
            mi = jnp.where(m, 1, 0).astype(jnp.int32)
            c = plsc.cumsum(mi)
            posk = jnp.where(m, starts[e] + c - 1, posk)
            starts[e] = starts[e] + jnp.sum(mi)
        pos_v[pl.ds(k * 16, 16)] = posk
    pltpu.sync_copy(pos_v, pos_hbm.at[pl.ds(base, TPT)])
    pltpu.sync_copy(xg_hbm.at[pl.ds(base, TPT)], rows_v)
    pltpu.async_copy(rows_v, xs_hbm.at[pos_v], sem).wait()


def _sort_scatter(eid, xg, hist):
    return pl.kernel(
        _sort_scatter_body,
        out_type=(
            jax.ShapeDtypeStruct((NPAD, D), jnp.float32),
            jax.ShapeDtypeStruct((A,), jnp.int32),
            jax.ShapeDtypeStruct((NBV * 16,), jnp.int32),
            jax.ShapeDtypeStruct((16,), jnp.int32),
            jax.ShapeDtypeStruct((16,), jnp.int32),
        ),
        mesh=plsc.VectorSubcoreMesh(**_SC_MESH),
        compiler_params=pltpu.CompilerParams(needs_layout_passes=False),
        scratch_types=[
            pltpu.VMEM((TPT,), jnp.int32),
            pltpu.VMEM((TPT, D), jnp.float32),
            pltpu.VMEM((TPT,), jnp.int32),
            pltpu.VMEM((NT, 16), jnp.int32),
            pltpu.VMEM((NBV * 16,), jnp.int32),
            pltpu.VMEM((16,), jnp.int32),
            pltpu.VMEM((16,), jnp.int32),
            pltpu.SemaphoreType.DMA,
        ],
    )(eid, xg, hist)


_LA = 2
_NBUF = 3


def _grouped_body(beid_ref, fo_ref, slot_ref, xs_ref, w1_hbm, w3_hbm, w2_hbm,
                  ys_ref, w1b, w3b, w2b, sems):
    i = pl.program_id(0)

    def w_copies(e_idx, s_idx):
        return (
            pltpu.make_async_copy(w1_hbm.at[e_idx], w1b.at[s_idx],
                                  sems.at[s_idx, 0]),
            pltpu.make_async_copy(w3_hbm.at[e_idx], w3b.at[s_idx],
                                  sems.at[s_idx, 1]),
            pltpu.make_async_copy(w2_hbm.at[e_idx], w2b.at[s_idx],
                                  sems.at[s_idx, 2]),
        )

    for ee in range(E):
        fo_e = fo_ref[ee]
        sl_e = slot_ref[ee]

        @pl.when((fo_e >= 0) & (i == jnp.maximum(fo_e - _LA, 0)))
        def _():
            for c in w_copies(ee, sl_e):
                c.start()

        @pl.when(fo_e == i)
        def _():
            for c in w_copies(ee, sl_e):
                c.wait()

    e = beid_ref[i]
    sl = slot_ref[e]
    xb = xs_ref[...].astype(jnp.bfloat16)
    w1c = w1b[pl.ds(sl, 1)][0].astype(jnp.bfloat16)
    w3c = w3b[pl.ds(sl, 1)][0].astype(jnp.bfloat16)
    w2c = w2b[pl.ds(sl, 1)][0].astype(jnp.bfloat16)
    h1 = jnp.dot(xb, w1c, preferred_element_type=jnp.float32)
    h3 = jnp.dot(xb, w3c, preferred_element_type=jnp.float32)
    mid = (h1 * jax.nn.sigmoid(h1) * h3).astype(jnp.bfloat16)
    ys_ref[...] = jnp.dot(mid, w2c, preferred_element_type=jnp.float32)


def _grouped(beid, fo, slot, xs, w1, w3, w2):
    grid_spec = pltpu.PrefetchScalarGridSpec(
        num_scalar_prefetch=3,
        grid=(NBP,),
        in_specs=[
            pl.BlockSpec((B, D), lambda i, be, fo_, sl: (i, 0)),
            pl.BlockSpec(memory_space=pltpu.MemorySpace.HBM),
            pl.BlockSpec(memory_space=pltpu.MemorySpace.HBM),
            pl.BlockSpec(memory_space=pltpu.MemorySpace.HBM),
        ],
        out_specs=pl.BlockSpec((B, D), lambda i, be, fo_, sl: (i, 0)),
        scratch_shapes=[
            pltpu.VMEM((_NBUF, D, F), jnp.float32),
            pltpu.VMEM((_NBUF, D, F), jnp.float32),
            pltpu.VMEM((_NBUF, F, D), jnp.float32),
            pltpu.SemaphoreType.DMA((_NBUF, 3)),
        ],
    )
    return pl.pallas_call(
        _grouped_body,
        grid_spec=grid_spec,
        out_shape=jax.ShapeDtypeStruct((NPAD, D), jnp.float32),
    )(beid, fo, slot, xs, w1, w3, w2)


def _merge_body(pos_hbm, shared_hbm, ys_hbm, out_hbm, pos_v, rows_v, sh_v, sem):
    wid = lax.axis_index("s") * NC + lax.axis_index("c")
    base = wid * TPT
    pltpu.sync_copy(pos_hbm.at[pl.ds(base, TPT)], pos_v)
    pltpu.async_copy(ys_hbm.at[pos_v], rows_v, sem).wait()
    pltpu.sync_copy(shared_hbm.at[pl.ds(base, TPT)], sh_v)

    def _add_row(r, carry):
        for c in range(CPR):
            rows_v[r, pl.ds(c * 16, 16)] = (rows_v[r, pl.ds(c * 16, 16)]
                                            + sh_v[r, pl.ds(c * 16, 16)])
        return carry

    lax.fori_loop(0, TPT, _add_row, 0)
    pltpu.sync_copy(rows_v, out_hbm.at[pl.ds(base, TPT)])


def _merge(pos, shared, ys):
    return pl.kernel(
        _merge_body,
        out_type=jax.ShapeDtypeStruct((A, D), jnp.float32),
        mesh=plsc.VectorSubcoreMesh(**_SC_MESH),
        compiler_params=pltpu.CompilerParams(needs_layout_passes=False),
        scratch_types=[
            pltpu.VMEM((TPT,), jnp.int32),
            pltpu.VMEM((TPT, D), jnp.float32),
            pltpu.VMEM((TPT, D), jnp.float32),
            pltpu.SemaphoreType.DMA,
        ],
    )(pos, shared, ys)


def kernel(x_bsD, router_DE, w1, w3, w2, sw1, sw3, sw2):
    b, s, d = x_bsD.shape
    x = x_bsD.reshape(-1, d)
    eid2, hist, xg = _router(x, router_DE)
    xs, pos, beid, fo, slot = _sort_scatter(eid2, xg, hist)
    shared = _shared(x, sw1, sw3, sw2)
    ys = _grouped(beid, fo, slot, xs, w1, w3, w2)
    out = _merge(pos, shared, ys)
    return out.reshape(b, s, d)
